# baseline (speedup 1.0000x reference)
import functools

import jax
import jax.numpy as jnp
from jax.experimental import pallas as pl
from jax.experimental.pallas import tpu as pltpu


def _compute_block(x, params_ref, inv_n, inv_nm1):
    mean_t = params_ref[0]
    std_t = params_ref[1]
    min_v = params_ref[2]
    max_v = params_ref[3]
    s = jnp.sum(x, axis=-1, keepdims=True)
    sq = jnp.sum(x * x, axis=-1, keepdims=True)
    mu = s * inv_n
    var = (sq - s * mu) * inv_nm1
    gain = std_t * jax.lax.rsqrt(var)
    shift = gain * (mean_t - mu)
    return jnp.minimum(jnp.maximum(x * gain + shift, min_v), max_v)


def _nc_manual_kernel(params_ref, x_hbm, o_hbm, xbuf, ybuf, in_sem, out_sem,
                      *, tb, g, inv_n, inv_nm1):
    i = pl.program_id(0)
    slot = jax.lax.rem(i, 2)
    nslot = 1 - slot

    def in_copy(blk, sl):
        return pltpu.make_async_copy(
            x_hbm.at[pl.ds(blk * tb, tb)], xbuf.at[sl], in_sem.at[sl])

    def out_copy(blk, sl):
        return pltpu.make_async_copy(
            ybuf.at[sl], o_hbm.at[pl.ds(blk * tb, tb)], out_sem.at[sl])

    @pl.when(i == 0)
    def _():
        in_copy(0, 0).start()

    @pl.when(i + 1 < g)
    def _():
        in_copy(i + 1, nslot).start()

    in_copy(i, slot).wait()

    x = xbuf[slot].astype(jnp.float32)
    y = _compute_block(x, params_ref, inv_n, inv_nm1)

    @pl.when(i >= 2)
    def _():
        out_copy(i - 2, slot).wait()

    ybuf[slot] = y.astype(ybuf.dtype)
    out_copy(i, slot).start()

    @pl.when(i == g - 1)
    def _():
        out_copy(i, slot).wait()
        if g >= 2:
            out_copy(i - 1, nslot).wait()


def _nc_fused_kernel(params_ref, x_ref, o_ref, *, inv_n, inv_nm1):
    x = x_ref[...].astype(jnp.float32)
    o_ref[...] = _compute_block(x, params_ref, inv_n, inv_nm1).astype(o_ref.dtype)


@jax.jit
def _normalize_clamp(x, mean, std, min_val, max_val):
    B, C, H, W = x.shape
    N = C * H * W
    x2d = x.reshape(B, N)

    params = jnp.stack([
        jnp.asarray(mean, jnp.float32), jnp.asarray(std, jnp.float32),
        jnp.asarray(min_val, jnp.float32), jnp.asarray(max_val, jnp.float32)])
    smem_spec = pl.BlockSpec(memory_space=pltpu.MemorySpace.SMEM)

    tb = 8
    if B % tb == 0 and B // tb >= 2:
        g = B // tb
        out2d = pl.pallas_call(
            functools.partial(_nc_manual_kernel, tb=tb, g=g,
                              inv_n=1.0 / N, inv_nm1=1.0 / (N - 1)),
            out_shape=jax.ShapeDtypeStruct((B, N), x.dtype),
            grid=(g,),
            in_specs=[smem_spec,
                      pl.BlockSpec(memory_space=pltpu.MemorySpace.HBM)],
            out_specs=pl.BlockSpec(memory_space=pltpu.MemorySpace.HBM),
            scratch_shapes=[
                pltpu.VMEM((2, tb, N), x.dtype),
                pltpu.VMEM((2, tb, N), x.dtype),
                pltpu.SemaphoreType.DMA((2,)),
                pltpu.SemaphoreType.DMA((2,)),
            ],
            compiler_params=pltpu.CompilerParams(
                dimension_semantics=("arbitrary",),
                vmem_limit_bytes=48 * 1024 * 1024),
        )(params, x2d)
    else:
        tb = B if B <= 8 else 8
        out2d = pl.pallas_call(
            functools.partial(_nc_fused_kernel,
                              inv_n=1.0 / N, inv_nm1=1.0 / (N - 1)),
            out_shape=jax.ShapeDtypeStruct((B, N), x.dtype),
            grid=(pl.cdiv(B, tb),),
            in_specs=[smem_spec, pl.BlockSpec((tb, N), lambda b: (b, 0))],
            out_specs=pl.BlockSpec((tb, N), lambda b: (b, 0)),
            compiler_params=pltpu.CompilerParams(
                dimension_semantics=("arbitrary",),
                vmem_limit_bytes=48 * 1024 * 1024),
        )(params, x2d)
    return out2d.reshape(B, C, H, W)


def kernel(x, mean, std, min_val, max_val):
    return _normalize_clamp(x, mean, std, min_val, max_val)
